# Initial kernel scaffold; baseline (speedup 1.0000x reference)
import jax
import jax.numpy as jnp
from jax.experimental import pallas as pl


def kernel(x, coord, edge_index, W1, b1, W2, b2, W3, b3, W4, b4, W5, b5):
    raise NotImplementedError("write your pallas kernel here")


# speedup vs baseline: 2.0193x; 2.0193x over previous
import functools

import jax
import jax.numpy as jnp
from jax import lax
from jax.experimental import pallas as pl
from jax.experimental.pallas import tpu as pltpu
from jax.experimental.pallas import tpu_sc as plsc

D = 512
CW = 128
TW = D + CW
CP = 16
NCORES = 2
NSUB = 16
NW = NCORES * NSUB


def _proj_body(x_ref, cp_ref, w1r_ref, w1c_ref, b1_ref, tr_ref, tc_ref):
    xb = x_ref[...]
    cpb = cp_ref[...]
    pr = jnp.dot(xb, w1r_ref[...],
                 preferred_element_type=jnp.float32) + b1_ref[...]
    pc = jnp.dot(xb, w1c_ref[...], preferred_element_type=jnp.float32)
    tr_ref[...] = jnp.concatenate([pr, cpb], axis=1)
    tc_ref[...] = jnp.concatenate([pc, -cpb], axis=1)


def _project(x, cp, w1r, w1c, b1):
    n = x.shape[0]
    bn = 1000
    return pl.pallas_call(
        _proj_body,
        grid=(n // bn,),
        in_specs=[
            pl.BlockSpec((bn, D), lambda i: (i, 0)),
            pl.BlockSpec((bn, CW), lambda i: (i, 0)),
            pl.BlockSpec((D, D), lambda i: (0, 0)),
            pl.BlockSpec((D, D), lambda i: (0, 0)),
            pl.BlockSpec((1, D), lambda i: (0, 0)),
        ],
        out_specs=[
            pl.BlockSpec((bn, TW), lambda i: (i, 0)),
            pl.BlockSpec((bn, TW), lambda i: (i, 0)),
        ],
        out_shape=[
            jax.ShapeDtypeStruct((n, TW), jnp.float32),
            jax.ShapeDtypeStruct((n, TW), jnp.float32),
        ],
    )(x, cp, w1r, w1c, b1.reshape(1, D))


def _gather_combine(tr, tc, row, col):
    e = row.shape[0]
    epw = e // NW
    cb = 40
    nch = epw // cb
    nv = (D + CP) // 16
    mesh = plsc.VectorSubcoreMesh(core_axis_name="c", subcore_axis_name="s")

    @functools.partial(
        pl.kernel,
        out_type=[jax.ShapeDtypeStruct((e, D), jnp.float32),
                  jax.ShapeDtypeStruct((e, CW), jnp.float32)],
        mesh=mesh,
        scratch_types=[
            pltpu.VMEM((cb,), jnp.int32),
            pltpu.VMEM((cb,), jnp.int32),
            pltpu.VMEM((cb, TW), jnp.float32),
            pltpu.VMEM((cb, TW), jnp.float32),
            pltpu.SemaphoreType.DMA,
        ],
    )
    def k(tr_hbm, tc_hbm, row_hbm, col_hbm, g_hbm, dx_hbm,
          idxr, idxc, bufa, bufb, sem):
        wid = lax.axis_index("s") * NCORES + lax.axis_index("c")
        base0 = wid * epw

        def chunk(ci, carry):
            base = base0 + ci * cb
            pltpu.sync_copy(row_hbm.at[pl.ds(base, cb)], idxr)
            pltpu.sync_copy(col_hbm.at[pl.ds(base, cb)], idxc)
            c0 = pltpu.async_copy(tr_hbm.at[idxr], bufa, sem)
            c1 = pltpu.async_copy(tc_hbm.at[idxc], bufb, sem)
            c0.wait()
            c1.wait()

            def rowbody(r, acc):
                for kk in range(nv):
                    sl = pl.ds(kk * 16, 16)
                    bufa[r, sl] = bufa[r, sl] + bufb[r, sl]
                return acc

            lax.fori_loop(0, cb, rowbody, 0)
            pltpu.sync_copy(bufa.at[pl.ds(0, cb), pl.ds(0, D)],
                            g_hbm.at[pl.ds(base, cb)])
            pltpu.sync_copy(bufa.at[pl.ds(0, cb), pl.ds(D, CW)],
                            dx_hbm.at[pl.ds(base, cb)])
            return carry

        lax.fori_loop(0, nch, chunk, 0)

    return k(tr, tc, row, col)


def _edge_body(g_ref, dx_ref, w1d_ref, w2_ref, b2_ref, w5_ref, b5_ref,
               msg_ref, cu_ref):
    g = g_ref[...]
    dx = dx_ref[...]
    dist = jnp.sqrt(jnp.sum(dx * dx, axis=1, keepdims=True))
    h = g + dist * w1d_ref[...]
    h = h * jax.nn.sigmoid(h)
    msg = jnp.dot(h, w2_ref[...], preferred_element_type=jnp.float32) \
        + b2_ref[...]
    msg_ref[...] = msg
    cw = jax.nn.sigmoid(
        jnp.sum(msg * w5_ref[...], axis=1, keepdims=True) + b5_ref[...])
    cu_ref[...] = dx * cw


def _edge_mlp(g, dx, w1d, w2, b2, w5t, b5):
    e = g.shape[0]
    be = 800
    return pl.pallas_call(
        _edge_body,
        grid=(e // be,),
        in_specs=[
            pl.BlockSpec((be, D), lambda i: (i, 0)),
            pl.BlockSpec((be, CW), lambda i: (i, 0)),
            pl.BlockSpec((1, D), lambda i: (0, 0)),
            pl.BlockSpec((D, D), lambda i: (0, 0)),
            pl.BlockSpec((1, D), lambda i: (0, 0)),
            pl.BlockSpec((1, D), lambda i: (0, 0)),
            pl.BlockSpec((1, 1), lambda i: (0, 0)),
        ],
        out_specs=[
            pl.BlockSpec((be, D), lambda i: (i, 0)),
            pl.BlockSpec((be, CW), lambda i: (i, 0)),
        ],
        out_shape=[
            jax.ShapeDtypeStruct((e, D), jnp.float32),
            jax.ShapeDtypeStruct((e, CW), jnp.float32),
        ],
    )(g, dx, w1d, w2, b2.reshape(1, D), w5t, b5.reshape(1, 1))


def _scatter_combine(msg, cu, row, np_):
    e = msg.shape[0]
    ept = e // NSUB
    bd = 80
    nch = ept // bd
    epth = e // NW
    bd2 = 40
    nch2 = epth // bd2
    rpt = np_ // NSUB
    fs = 128
    zr = 128
    mesh = plsc.VectorSubcoreMesh(core_axis_name="c", subcore_axis_name="s")

    @functools.partial(
        pl.kernel,
        out_type=[jax.ShapeDtypeStruct((np_, D), jnp.float32),
                  jax.ShapeDtypeStruct((2, np_, CW), jnp.float32)],
        mesh=mesh,
        scratch_types=[
            pltpu.VMEM((bd,), jnp.int32),
            pltpu.VMEM((bd2,), jnp.int32),
            pltpu.VMEM((bd, fs), jnp.float32),
            pltpu.VMEM((bd2, fs), jnp.float32),
            pltpu.VMEM((zr, fs), jnp.float32),
            pltpu.VMEM_SHARED((np_, fs), jnp.float32),
            pltpu.SemaphoreType.DMA,
        ],
    )
    def k(msg_hbm, cu_hbm, row_hbm, nm_hbm, dc_hbm,
          idx, idx2, mbuf, cbuf, zbuf, acc, sem):
        core = lax.axis_index("c")
        tid = lax.axis_index("s")
        r0 = tid * rpt

        def zb(i, c):
            for kk in range(fs // 16):
                zbuf[i, pl.ds(kk * 16, 16)] = jnp.zeros((16,), jnp.float32)
            return c
        lax.fori_loop(0, zr, zb, 0)

        def zero_acc():
            for q in range(rpt // zr):
                pltpu.sync_copy(zbuf, acc.at[pl.ds(r0 + q * zr, zr)])

        zero_acc()
        plsc.subcore_barrier()

        for p in range(2):
            joff = (core * 2 + p) * fs

            def chunk(ci, carry):
                base = tid * ept + ci * bd
                pltpu.sync_copy(row_hbm.at[pl.ds(base, bd)], idx)
                pltpu.sync_copy(
                    msg_hbm.at[pl.ds(base, bd), pl.ds(joff, fs)], mbuf)
                pltpu.sync_copy(mbuf, acc.at[idx], add=True)
                return carry

            lax.fori_loop(0, nch, chunk, 0)
            plsc.subcore_barrier()
            pltpu.sync_copy(acc.at[pl.ds(r0, rpt)],
                            nm_hbm.at[pl.ds(r0, rpt), pl.ds(joff, fs)])
            zero_acc()
            plsc.subcore_barrier()

        ebase0 = core * (e // 2) + tid * epth

        def chunk2(ci, carry):
            base = ebase0 + ci * bd2
            pltpu.sync_copy(row_hbm.at[pl.ds(base, bd2)], idx2)
            pltpu.sync_copy(cu_hbm.at[pl.ds(base, bd2)], cbuf)
            pltpu.sync_copy(cbuf, acc.at[idx2], add=True)
            return carry

        lax.fori_loop(0, nch2, chunk2, 0)
        plsc.subcore_barrier()
        pltpu.sync_copy(acc.at[pl.ds(r0, rpt)],
                        dc_hbm.at[core, pl.ds(r0, rpt)])

    return k(msg, cu, row)


def _node_body(nm_ref, x_ref, w3_ref, b3_ref, w4_ref, b4_ref,
               cp_ref, dc_ref, xo_ref, co_ref):
    t = jnp.dot(nm_ref[...], w3_ref[...],
                preferred_element_type=jnp.float32) + b3_ref[...]
    t = t * jax.nn.sigmoid(t)
    xo_ref[...] = x_ref[...] + jnp.dot(
        t, w4_ref[...], preferred_element_type=jnp.float32) + b4_ref[...]
    co_ref[...] = cp_ref[...] + dc_ref[0] + dc_ref[1]


def _node_mlp(nm, x, w3, b3, w4, b4, cp, dc):
    n = x.shape[0]
    bn = 1000
    return pl.pallas_call(
        _node_body,
        grid=(n // bn,),
        in_specs=[
            pl.BlockSpec((bn, D), lambda i: (i, 0)),
            pl.BlockSpec((bn, D), lambda i: (i, 0)),
            pl.BlockSpec((D, D), lambda i: (0, 0)),
            pl.BlockSpec((1, D), lambda i: (0, 0)),
            pl.BlockSpec((D, D), lambda i: (0, 0)),
            pl.BlockSpec((1, D), lambda i: (0, 0)),
            pl.BlockSpec((bn, CW), lambda i: (i, 0)),
            pl.BlockSpec((2, bn, CW), lambda i: (0, i, 0)),
        ],
        out_specs=[
            pl.BlockSpec((bn, D), lambda i: (i, 0)),
            pl.BlockSpec((bn, CW), lambda i: (i, 0)),
        ],
        out_shape=[
            jax.ShapeDtypeStruct((n, D), jnp.float32),
            jax.ShapeDtypeStruct((n, CW), jnp.float32),
        ],
    )(nm, x, w3, b3.reshape(1, D), w4, b4.reshape(1, D), cp, dc)


def kernel(x, coord, edge_index, W1, b1, W2, b2, W3, b3, W4, b4, W5, b5):
    n, d = x.shape
    row = edge_index[0].astype(jnp.int32)
    col = edge_index[1].astype(jnp.int32)
    w1r = W1[:d]
    w1c = W1[d:2 * d]
    w1d = W1[2 * d].reshape(1, d)
    cpw = jnp.pad(coord, ((0, 0), (0, CW - 3)))
    tr, tc = _project(x, cpw, w1r, w1c, b1)
    g, dx = _gather_combine(tr, tc, row, col)
    msg, cu = _edge_mlp(g, dx, w1d, W2, b2, W5.reshape(1, d), b5)
    np_ = ((n + 2047) // 2048) * 2048
    nm, dc = _scatter_combine(msg, cu, row, np_)
    x_out, co = _node_mlp(nm, x, W3, b3, W4, b4, cpw, dc)
    return (x_out, co[:, :3])


# speedup vs baseline: 2.6251x; 1.3000x over previous
import functools

import jax
import jax.numpy as jnp
from jax import lax
from jax.experimental import pallas as pl
from jax.experimental.pallas import tpu as pltpu
from jax.experimental.pallas import tpu_sc as plsc

D = 512
CW = 128
TW = D + CW
CP = 16
NCORES = 2
NSUB = 16
NW = NCORES * NSUB


def _proj_body(x_ref, cp_ref, w1r_ref, w1c_ref, b1_ref, tr_ref, tc_ref):
    xb = x_ref[...]
    cpb = cp_ref[...]
    pr = jnp.dot(xb, w1r_ref[...],
                 preferred_element_type=jnp.float32) + b1_ref[...]
    pc = jnp.dot(xb, w1c_ref[...], preferred_element_type=jnp.float32)
    tr_ref[...] = jnp.concatenate([pr, cpb], axis=1)
    tc_ref[...] = jnp.concatenate([pc, -cpb], axis=1)


def _project(x, cp, w1r, w1c, b1):
    n = x.shape[0]
    bn = 1000
    return pl.pallas_call(
        _proj_body,
        grid=(n // bn,),
        in_specs=[
            pl.BlockSpec((bn, D), lambda i: (i, 0)),
            pl.BlockSpec((bn, CW), lambda i: (i, 0)),
            pl.BlockSpec((D, D), lambda i: (0, 0)),
            pl.BlockSpec((D, D), lambda i: (0, 0)),
            pl.BlockSpec((1, D), lambda i: (0, 0)),
        ],
        out_specs=[
            pl.BlockSpec((bn, TW), lambda i: (i, 0)),
            pl.BlockSpec((bn, TW), lambda i: (i, 0)),
        ],
        out_shape=[
            jax.ShapeDtypeStruct((n, TW), jnp.float32),
            jax.ShapeDtypeStruct((n, TW), jnp.float32),
        ],
    )(x, cp, w1r, w1c, b1.reshape(1, D))


def _gather_combine(tr, tc, row, col):
    e = row.shape[0]
    epw = e // NW
    cb = 40
    nch = epw // cb
    nv = (D + CP) // 16
    mesh = plsc.VectorSubcoreMesh(core_axis_name="c", subcore_axis_name="s")

    @functools.partial(
        pl.kernel,
        out_type=[jax.ShapeDtypeStruct((e, D), jnp.float32),
                  jax.ShapeDtypeStruct((e, CW), jnp.float32)],
        mesh=mesh,
        scratch_types=[
            pltpu.VMEM((cb,), jnp.int32),
            pltpu.VMEM((cb,), jnp.int32),
            pltpu.VMEM((cb,), jnp.int32),
            pltpu.VMEM((cb,), jnp.int32),
            pltpu.VMEM((cb, TW), jnp.float32),
            pltpu.VMEM((cb, TW), jnp.float32),
            pltpu.VMEM((cb, TW), jnp.float32),
            pltpu.VMEM((cb, TW), jnp.float32),
            pltpu.SemaphoreType.DMA,
            pltpu.SemaphoreType.DMA,
            pltpu.SemaphoreType.DMA,
            pltpu.SemaphoreType.DMA,
        ],
    )
    def k(tr_hbm, tc_hbm, row_hbm, col_hbm, g_hbm, dx_hbm,
          idxr0, idxr1, idxc0, idxc1, ba0, ba1, bb0, bb1,
          gs0, gs1, ws0, ws1):
        idxr = (idxr0, idxr1)
        idxc = (idxc0, idxc1)
        ba = (ba0, ba1)
        bb = (bb0, bb1)
        gs = (gs0, gs1)
        ws = (ws0, ws1)
        wid = lax.axis_index("s") * NCORES + lax.axis_index("c")
        base0 = wid * epw

        def gfire(ci, s):
            base = base0 + ci * cb
            pltpu.sync_copy(row_hbm.at[pl.ds(base, cb)], idxr[s])
            pltpu.sync_copy(col_hbm.at[pl.ds(base, cb)], idxc[s])
            pltpu.async_copy(tr_hbm.at[idxr[s]], ba[s], gs[s])
            pltpu.async_copy(tc_hbm.at[idxc[s]], bb[s], gs[s])

        def gwait(s):
            pltpu.make_async_copy(tr_hbm.at[idxr[s]], ba[s], gs[s]).wait()
            pltpu.make_async_copy(tc_hbm.at[idxc[s]], bb[s], gs[s]).wait()

        def add(s):
            def rowbody(r, acc):
                for kk in range(nv):
                    sl = pl.ds(kk * 16, 16)
                    ba[s][r, sl] = ba[s][r, sl] + bb[s][r, sl]
                return acc
            lax.fori_loop(0, cb, rowbody, 0)

        def wfire(ci, s):
            base = base0 + ci * cb
            pltpu.async_copy(ba[s].at[pl.ds(0, cb), pl.ds(0, D)],
                             g_hbm.at[pl.ds(base, cb)], ws[s])
            pltpu.async_copy(ba[s].at[pl.ds(0, cb), pl.ds(D, CW)],
                             dx_hbm.at[pl.ds(base, cb)], ws[s])

        def wwait(s):
            pltpu.make_async_copy(ba[s].at[pl.ds(0, cb), pl.ds(0, D)],
                                  g_hbm.at[pl.ds(base0, cb)], ws[s]).wait()
            pltpu.make_async_copy(ba[s].at[pl.ds(0, cb), pl.ds(D, CW)],
                                  dx_hbm.at[pl.ds(base0, cb)], ws[s]).wait()

        gfire(0, 0)

        def pair(pi, carry):
            for b in range(2):
                ci = 2 * pi + b
                s = b
                so = 1 - b

                @pl.when(ci > 0)
                def _():
                    wwait(so)

                gfire(ci + 1, so)
                gwait(s)
                add(s)
                wfire(ci, s)
            return carry

        lax.fori_loop(0, (nch - 1) // 2, pair, 0)
        wwait(1)
        gwait(0)
        add(0)
        wfire(nch - 1, 0)
        wwait(0)

    return k(tr, tc, row, col)


def _edge_body(g_ref, dx_ref, w1d_ref, w2_ref, b2_ref, w5_ref, b5_ref,
               msg_ref, cu_ref):
    g = g_ref[...]
    dx = dx_ref[...]
    dist = jnp.sqrt(jnp.sum(dx * dx, axis=1, keepdims=True))
    h = g + dist * w1d_ref[...]
    h = h * jax.nn.sigmoid(h)
    msg = jnp.dot(h, w2_ref[...], preferred_element_type=jnp.float32) \
        + b2_ref[...]
    msg_ref[...] = msg
    cw = jax.nn.sigmoid(
        jnp.sum(msg * w5_ref[...], axis=1, keepdims=True) + b5_ref[...])
    cu_ref[...] = dx * cw


def _edge_mlp(g, dx, w1d, w2, b2, w5t, b5):
    e = g.shape[0]
    be = 800
    return pl.pallas_call(
        _edge_body,
        grid=(e // be,),
        in_specs=[
            pl.BlockSpec((be, D), lambda i: (i, 0)),
            pl.BlockSpec((be, CW), lambda i: (i, 0)),
            pl.BlockSpec((1, D), lambda i: (0, 0)),
            pl.BlockSpec((D, D), lambda i: (0, 0)),
            pl.BlockSpec((1, D), lambda i: (0, 0)),
            pl.BlockSpec((1, D), lambda i: (0, 0)),
            pl.BlockSpec((1, 1), lambda i: (0, 0)),
        ],
        out_specs=[
            pl.BlockSpec((be, D), lambda i: (i, 0)),
            pl.BlockSpec((be, CW), lambda i: (i, 0)),
        ],
        out_shape=[
            jax.ShapeDtypeStruct((e, D), jnp.float32),
            jax.ShapeDtypeStruct((e, CW), jnp.float32),
        ],
    )(g, dx, w1d, w2, b2.reshape(1, D), w5t, b5.reshape(1, 1))


def _scatter_combine(msg, cu, row, np_):
    e = msg.shape[0]
    ept = e // NSUB
    bd = 80
    nch = ept // bd
    epth = e // NW
    bd2 = 40
    nch2 = epth // bd2
    rpt = np_ // NSUB
    fs = 128
    zr = 128
    mesh = plsc.VectorSubcoreMesh(core_axis_name="c", subcore_axis_name="s")

    @functools.partial(
        pl.kernel,
        out_type=[jax.ShapeDtypeStruct((np_, D), jnp.float32),
                  jax.ShapeDtypeStruct((2, np_, CW), jnp.float32)],
        mesh=mesh,
        scratch_types=[
            pltpu.VMEM((bd,), jnp.int32),
            pltpu.VMEM((bd,), jnp.int32),
            pltpu.VMEM((bd2,), jnp.int32),
            pltpu.VMEM((bd2,), jnp.int32),
            pltpu.VMEM((bd, fs), jnp.float32),
            pltpu.VMEM((bd, fs), jnp.float32),
            pltpu.VMEM((bd2, fs), jnp.float32),
            pltpu.VMEM((bd2, fs), jnp.float32),
            pltpu.VMEM((zr, fs), jnp.float32),
            pltpu.VMEM_SHARED((np_, fs), jnp.float32),
            pltpu.SemaphoreType.DMA,
            pltpu.SemaphoreType.DMA,
            pltpu.SemaphoreType.DMA,
            pltpu.SemaphoreType.DMA,
        ],
    )
    def k(msg_hbm, cu_hbm, row_hbm, nm_hbm, dc_hbm,
          idxa0, idxa1, idxb0, idxb1, mb0, mb1, cb0, cb1, zbuf, acc,
          ls0, ls1, ss0, ss1):
        core = lax.axis_index("c")
        tid = lax.axis_index("s")
        r0 = tid * rpt
        ls = (ls0, ls1)
        ss = (ss0, ss1)

        def zb(i, c):
            for kk in range(fs // 16):
                zbuf[i, pl.ds(kk * 16, 16)] = jnp.zeros((16,), jnp.float32)
            return c
        lax.fori_loop(0, zr, zb, 0)

        def zero_acc():
            for q in range(rpt // zr):
                pltpu.sync_copy(zbuf, acc.at[pl.ds(r0 + q * zr, zr)])

        def scatter_pass(src_hbm, joff, w, ebase0, nchp, idxp, buf):
            def lfire(ci, s):
                base = ebase0 + ci * w[0]
                pltpu.sync_copy(row_hbm.at[pl.ds(base, w[0])], idxp[s])
                pltpu.async_copy(
                    src_hbm.at[pl.ds(base, w[0]), pl.ds(joff, fs)],
                    buf[s], ls[s])

            def lwait(s):
                pltpu.make_async_copy(
                    src_hbm.at[pl.ds(ebase0, w[0]), pl.ds(joff, fs)],
                    buf[s], ls[s]).wait()

            def sfire(s):
                pltpu.async_copy(buf[s], acc.at[idxp[s]], ss[s], add=True)

            def swait(s):
                pltpu.make_async_copy(buf[s], acc.at[idxp[s]], ss[s]).wait()

            lfire(0, 0)

            def pair(pi, carry):
                for b in range(2):
                    ci = 2 * pi + b
                    s = b
                    so = 1 - b
                    lwait(s)
                    sfire(s)

                    @pl.when(ci > 0)
                    def _():
                        swait(so)

                    lfire(ci + 1, so)
                return carry

            lax.fori_loop(0, (nchp - 1) // 2, pair, 0)
            lwait(0)
            sfire(0)
            swait(1)
            swait(0)

        zero_acc()
        plsc.subcore_barrier()

        for p in range(2):
            joff = (core * 2 + p) * fs
            scatter_pass(msg_hbm, joff, (bd,), tid * ept, nch,
                         (idxa0, idxa1), (mb0, mb1))
            plsc.subcore_barrier()
            pltpu.sync_copy(acc.at[pl.ds(r0, rpt)],
                            nm_hbm.at[pl.ds(r0, rpt), pl.ds(joff, fs)])
            zero_acc()
            plsc.subcore_barrier()

        scatter_pass(cu_hbm, 0, (bd2,), core * (e // 2) + tid * epth, nch2,
                     (idxb0, idxb1), (cb0, cb1))
        plsc.subcore_barrier()
        pltpu.sync_copy(acc.at[pl.ds(r0, rpt)],
                        dc_hbm.at[core, pl.ds(r0, rpt)])

    return k(msg, cu, row)


def _node_body(nm_ref, x_ref, w3_ref, b3_ref, w4_ref, b4_ref,
               cp_ref, dc_ref, xo_ref, co_ref):
    t = jnp.dot(nm_ref[...], w3_ref[...],
                preferred_element_type=jnp.float32) + b3_ref[...]
    t = t * jax.nn.sigmoid(t)
    xo_ref[...] = x_ref[...] + jnp.dot(
        t, w4_ref[...], preferred_element_type=jnp.float32) + b4_ref[...]
    co_ref[...] = cp_ref[...] + dc_ref[0] + dc_ref[1]


def _node_mlp(nm, x, w3, b3, w4, b4, cp, dc):
    n = x.shape[0]
    bn = 1000
    return pl.pallas_call(
        _node_body,
        grid=(n // bn,),
        in_specs=[
            pl.BlockSpec((bn, D), lambda i: (i, 0)),
            pl.BlockSpec((bn, D), lambda i: (i, 0)),
            pl.BlockSpec((D, D), lambda i: (0, 0)),
            pl.BlockSpec((1, D), lambda i: (0, 0)),
            pl.BlockSpec((D, D), lambda i: (0, 0)),
            pl.BlockSpec((1, D), lambda i: (0, 0)),
            pl.BlockSpec((bn, CW), lambda i: (i, 0)),
            pl.BlockSpec((2, bn, CW), lambda i: (0, i, 0)),
        ],
        out_specs=[
            pl.BlockSpec((bn, D), lambda i: (i, 0)),
            pl.BlockSpec((bn, CW), lambda i: (i, 0)),
        ],
        out_shape=[
            jax.ShapeDtypeStruct((n, D), jnp.float32),
            jax.ShapeDtypeStruct((n, CW), jnp.float32),
        ],
    )(nm, x, w3, b3.reshape(1, D), w4, b4.reshape(1, D), cp, dc)


def kernel(x, coord, edge_index, W1, b1, W2, b2, W3, b3, W4, b4, W5, b5):
    n, d = x.shape
    row = edge_index[0].astype(jnp.int32)
    col = edge_index[1].astype(jnp.int32)
    w1r = W1[:d]
    w1c = W1[d:2 * d]
    w1d = W1[2 * d].reshape(1, d)
    cpw = jnp.pad(coord, ((0, 0), (0, CW - 3)))
    tr, tc = _project(x, cpw, w1r, w1c, b1)
    g, dx = _gather_combine(tr, tc, row, col)
    msg, cu = _edge_mlp(g, dx, w1d, W2, b2, W5.reshape(1, d), b5)
    np_ = ((n + 2047) // 2048) * 2048
    nm, dc = _scatter_combine(msg, cu, row, np_)
    x_out, co = _node_mlp(nm, x, W3, b3, W4, b4, cpw, dc)
    return (x_out, co[:, :3])


# speedup vs baseline: 2.7553x; 1.0496x over previous
import functools

import jax
import jax.numpy as jnp
from jax import lax
from jax.experimental import pallas as pl
from jax.experimental.pallas import tpu as pltpu
from jax.experimental.pallas import tpu_sc as plsc

D = 512
CW = 128
DI = D // 2
WI = DI + CW
CP = 16
NCORES = 2
NSUB = 16
NW = NCORES * NSUB


def _proj_body(x_ref, w1r_ref, w1c_ref, b1_ref, pr_ref, pc_ref):
    xb = x_ref[...]
    pr = jnp.dot(xb, w1r_ref[...],
                 preferred_element_type=jnp.float32) + b1_ref[...]
    pc = jnp.dot(xb, w1c_ref[...], preferred_element_type=jnp.float32)
    pr_ref[...] = pr.astype(jnp.bfloat16)
    pc_ref[...] = pc.astype(jnp.bfloat16)


def _project(x, w1r, w1c, b1):
    n = x.shape[0]
    bn = 1000
    return pl.pallas_call(
        _proj_body,
        grid=(n // bn,),
        in_specs=[
            pl.BlockSpec((bn, D), lambda i: (i, 0)),
            pl.BlockSpec((D, D), lambda i: (0, 0)),
            pl.BlockSpec((D, D), lambda i: (0, 0)),
            pl.BlockSpec((1, D), lambda i: (0, 0)),
        ],
        out_specs=[
            pl.BlockSpec((bn, D), lambda i: (i, 0)),
            pl.BlockSpec((bn, D), lambda i: (i, 0)),
        ],
        out_shape=[
            jax.ShapeDtypeStruct((n, D), jnp.bfloat16),
            jax.ShapeDtypeStruct((n, D), jnp.bfloat16),
        ],
    )(x, w1r, w1c, b1.reshape(1, D))


def _pack_bf16(v):
    k = v.shape[-1] // 2
    pairs = jnp.stack([v[..., :k], v[..., k:]], axis=-1)
    return jax.lax.bitcast_convert_type(pairs, jnp.int32)


def _gather_combine(trf, tcf, cpw, ncpw, row, col):
    e = row.shape[0]
    epw = e // NW
    cb = 40
    nch = epw // cb
    mesh = plsc.VectorSubcoreMesh(core_axis_name="c", subcore_axis_name="s")

    @functools.partial(
        pl.kernel,
        out_type=[jax.ShapeDtypeStruct((e // cb, cb, DI), jnp.int32),
                  jax.ShapeDtypeStruct((e // cb, cb, DI), jnp.int32),
                  jax.ShapeDtypeStruct((e // cb, cb, CW), jnp.float32)],
        mesh=mesh,
        scratch_types=[
            pltpu.VMEM((cb,), jnp.int32),
            pltpu.VMEM((cb,), jnp.int32),
            pltpu.VMEM((cb,), jnp.int32),
            pltpu.VMEM((cb,), jnp.int32),
            pltpu.VMEM((cb, DI), jnp.int32),
            pltpu.VMEM((cb, DI), jnp.int32),
            pltpu.VMEM((cb, DI), jnp.int32),
            pltpu.VMEM((cb, DI), jnp.int32),
            pltpu.VMEM((cb, CW), jnp.float32),
            pltpu.VMEM((cb, CW), jnp.float32),
            pltpu.VMEM((cb, CW), jnp.float32),
            pltpu.VMEM((cb, CW), jnp.float32),
            pltpu.SemaphoreType.DMA,
            pltpu.SemaphoreType.DMA,
            pltpu.SemaphoreType.DMA,
            pltpu.SemaphoreType.DMA,
        ],
    )
    def k(trf_hbm, tcf_hbm, cpw_hbm, ncpw_hbm, row_hbm, col_hbm,
          gr_hbm, gc_hbm, dx_hbm,
          idxr0, idxr1, idxc0, idxc1, br0, br1, bc0, bc1,
          cr0, cr1, cc0, cc1, gs0, gs1, ws0, ws1):
        idxr = (idxr0, idxr1)
        idxc = (idxc0, idxc1)
        br = (br0, br1)
        bc = (bc0, bc1)
        cr = (cr0, cr1)
        cc = (cc0, cc1)
        gs = (gs0, gs1)
        ws = (ws0, ws1)
        wid = lax.axis_index("s") * NCORES + lax.axis_index("c")
        base0 = wid * epw

        def gfire(ci, s):
            base = base0 + ci * cb
            pltpu.sync_copy(row_hbm.at[pl.ds(base, cb)], idxr[s])
            pltpu.sync_copy(col_hbm.at[pl.ds(base, cb)], idxc[s])
            pltpu.async_copy(trf_hbm.at[idxr[s]], br[s], gs[s])
            pltpu.async_copy(tcf_hbm.at[idxc[s]], bc[s], gs[s])
            pltpu.async_copy(cpw_hbm.at[idxr[s]], cr[s], gs[s])
            pltpu.async_copy(ncpw_hbm.at[idxc[s]], cc[s], gs[s])

        def gwait(s):
            pltpu.make_async_copy(trf_hbm.at[idxr[s]], br[s], gs[s]).wait()
            pltpu.make_async_copy(tcf_hbm.at[idxc[s]], bc[s], gs[s]).wait()
            pltpu.make_async_copy(cpw_hbm.at[idxr[s]], cr[s], gs[s]).wait()
            pltpu.make_async_copy(ncpw_hbm.at[idxc[s]], cc[s], gs[s]).wait()

        def add(s):
            def rowbody(r, acc):
                sl = pl.ds(0, 16)
                cr[s][r, sl] = cr[s][r, sl] + cc[s][r, sl]
                return acc
            lax.fori_loop(0, cb, rowbody, 0)

        def wfire(ci, s):
            gci = wid * nch + ci
            pltpu.async_copy(br[s], gr_hbm.at[gci], ws[s])
            pltpu.async_copy(bc[s], gc_hbm.at[gci], ws[s])
            pltpu.async_copy(cr[s], dx_hbm.at[gci], ws[s])

        def wwait(s):
            pltpu.make_async_copy(br[s], gr_hbm.at[0], ws[s]).wait()
            pltpu.make_async_copy(bc[s], gc_hbm.at[0], ws[s]).wait()
            pltpu.make_async_copy(cr[s], dx_hbm.at[0], ws[s]).wait()

        gfire(0, 0)

        def pair(pi, carry):
            for b in range(2):
                ci = 2 * pi + b
                s = b
                so = 1 - b

                @pl.when(ci > 0)
                def _():
                    wwait(so)

                gfire(ci + 1, so)
                gwait(s)
                add(s)
                wfire(ci, s)
            return carry

        lax.fori_loop(0, (nch - 1) // 2, pair, 0)
        wwait(1)
        gwait(0)
        add(0)
        wfire(nch - 1, 0)
        wwait(0)

    return k(trf, tcf, cpw, ncpw, row, col)


def _unpack_add(wr, wc):
    lo = jax.lax.bitcast_convert_type(wr << 16, jnp.float32) \
        + jax.lax.bitcast_convert_type(wc << 16, jnp.float32)
    hi = jax.lax.bitcast_convert_type(wr & jnp.int32(-65536), jnp.float32) \
        + jax.lax.bitcast_convert_type(wc & jnp.int32(-65536), jnp.float32)
    return jnp.concatenate([lo, hi], axis=-1)


def _edge_body(gr_ref, gc_ref, dx_ref, w1d_ref, w2_ref, b2_ref, w5_ref,
               b5_ref, msg_ref, cu_ref):
    bc, cb, _ = gr_ref.shape
    be = bc * cb
    gb = _unpack_add(gr_ref[...], gc_ref[...]).reshape(be, D)
    dx = dx_ref[...].reshape(be, CW)
    dist = jnp.sqrt(jnp.sum(dx * dx, axis=1, keepdims=True))
    h = gb + dist * w1d_ref[...]
    h = h * jax.nn.sigmoid(h)
    msg = jnp.dot(h.astype(jnp.bfloat16), w2_ref[...],
                  preferred_element_type=jnp.float32) + b2_ref[...]
    msg_ref[...] = msg
    cw = jax.nn.sigmoid(
        jnp.sum(msg * w5_ref[...], axis=1, keepdims=True) + b5_ref[...])
    cu_ref[...] = dx * cw


def _edge_mlp(gr3, gc3, dx3, w1d, w2, b2, w5t, b5):
    nchk, cb, _ = gr3.shape
    e = nchk * cb
    be = 800
    bc = be // cb
    return pl.pallas_call(
        _edge_body,
        grid=(e // be,),
        in_specs=[
            pl.BlockSpec((bc, cb, DI), lambda i: (i, 0, 0)),
            pl.BlockSpec((bc, cb, DI), lambda i: (i, 0, 0)),
            pl.BlockSpec((bc, cb, CW), lambda i: (i, 0, 0)),
            pl.BlockSpec((1, D), lambda i: (0, 0)),
            pl.BlockSpec((D, D), lambda i: (0, 0)),
            pl.BlockSpec((1, D), lambda i: (0, 0)),
            pl.BlockSpec((1, D), lambda i: (0, 0)),
            pl.BlockSpec((1, 1), lambda i: (0, 0)),
        ],
        out_specs=[
            pl.BlockSpec((be, D), lambda i: (i, 0)),
            pl.BlockSpec((be, CW), lambda i: (i, 0)),
        ],
        out_shape=[
            jax.ShapeDtypeStruct((e, D), jnp.float32),
            jax.ShapeDtypeStruct((e, CW), jnp.float32),
        ],
    )(gr3, gc3, dx3, w1d, w2.astype(jnp.bfloat16), b2.reshape(1, D), w5t,
      b5.reshape(1, 1))


def _scatter_combine(msg, cu, row, np_):
    e = msg.shape[0]
    ept = e // NSUB
    bd = 80
    nch = ept // bd
    epth = e // NW
    bd2 = 40
    nch2 = epth // bd2
    rpt = np_ // NSUB
    fs = 128
    zr = 128
    mesh = plsc.VectorSubcoreMesh(core_axis_name="c", subcore_axis_name="s")

    @functools.partial(
        pl.kernel,
        out_type=[jax.ShapeDtypeStruct((np_, D), jnp.float32),
                  jax.ShapeDtypeStruct((2, np_, CW), jnp.float32)],
        mesh=mesh,
        scratch_types=[
            pltpu.VMEM((bd,), jnp.int32),
            pltpu.VMEM((bd,), jnp.int32),
            pltpu.VMEM((bd2,), jnp.int32),
            pltpu.VMEM((bd2,), jnp.int32),
            pltpu.VMEM((bd, fs), jnp.float32),
            pltpu.VMEM((bd, fs), jnp.float32),
            pltpu.VMEM((bd2, fs), jnp.float32),
            pltpu.VMEM((bd2, fs), jnp.float32),
            pltpu.VMEM((zr, fs), jnp.float32),
            pltpu.VMEM_SHARED((np_, fs), jnp.float32),
            pltpu.SemaphoreType.DMA,
            pltpu.SemaphoreType.DMA,
            pltpu.SemaphoreType.DMA,
            pltpu.SemaphoreType.DMA,
        ],
    )
    def k(msg_hbm, cu_hbm, row_hbm, nm_hbm, dc_hbm,
          idxa0, idxa1, idxb0, idxb1, mb0, mb1, cb0, cb1, zbuf, acc,
          ls0, ls1, ss0, ss1):
        core = lax.axis_index("c")
        tid = lax.axis_index("s")
        r0 = tid * rpt
        ls = (ls0, ls1)
        ss = (ss0, ss1)

        def zb(i, c):
            for kk in range(fs // 16):
                zbuf[i, pl.ds(kk * 16, 16)] = jnp.zeros((16,), jnp.float32)
            return c
        lax.fori_loop(0, zr, zb, 0)

        def zero_acc():
            for q in range(rpt // zr):
                pltpu.sync_copy(zbuf, acc.at[pl.ds(r0 + q * zr, zr)])

        def scatter_pass(src_hbm, joff, w, ebase0, nchp, idxp, buf):
            def lfire(ci, s):
                base = ebase0 + ci * w[0]
                pltpu.sync_copy(row_hbm.at[pl.ds(base, w[0])], idxp[s])
                pltpu.async_copy(
                    src_hbm.at[pl.ds(base, w[0]), pl.ds(joff, fs)],
                    buf[s], ls[s])

            def lwait(s):
                pltpu.make_async_copy(
                    src_hbm.at[pl.ds(ebase0, w[0]), pl.ds(joff, fs)],
                    buf[s], ls[s]).wait()

            def sfire(s):
                pltpu.async_copy(buf[s], acc.at[idxp[s]], ss[s], add=True)

            def swait(s):
                pltpu.make_async_copy(buf[s], acc.at[idxp[s]], ss[s]).wait()

            lfire(0, 0)

            def pair(pi, carry):
                for b in range(2):
                    ci = 2 * pi + b
                    s = b
                    so = 1 - b
                    lwait(s)
                    sfire(s)

                    @pl.when(ci > 0)
                    def _():
                        swait(so)

                    lfire(ci + 1, so)
                return carry

            lax.fori_loop(0, (nchp - 1) // 2, pair, 0)
            lwait(0)
            sfire(0)
            swait(1)
            swait(0)

        zero_acc()
        plsc.subcore_barrier()

        for p in range(2):
            joff = (core * 2 + p) * fs
            scatter_pass(msg_hbm, joff, (bd,), tid * ept, nch,
                         (idxa0, idxa1), (mb0, mb1))
            plsc.subcore_barrier()
            pltpu.sync_copy(acc.at[pl.ds(r0, rpt)],
                            nm_hbm.at[pl.ds(r0, rpt), pl.ds(joff, fs)])
            zero_acc()
            plsc.subcore_barrier()

        scatter_pass(cu_hbm, 0, (bd2,), core * (e // 2) + tid * epth, nch2,
                     (idxb0, idxb1), (cb0, cb1))
        plsc.subcore_barrier()
        pltpu.sync_copy(acc.at[pl.ds(r0, rpt)],
                        dc_hbm.at[core, pl.ds(r0, rpt)])

    return k(msg, cu, row)


def _node_body(nm_ref, x_ref, w3_ref, b3_ref, w4_ref, b4_ref,
               cp_ref, dc_ref, xo_ref, co_ref):
    t = jnp.dot(nm_ref[...], w3_ref[...],
                preferred_element_type=jnp.float32) + b3_ref[...]
    t = t * jax.nn.sigmoid(t)
    xo_ref[...] = x_ref[...] + jnp.dot(
        t, w4_ref[...], preferred_element_type=jnp.float32) + b4_ref[...]
    co_ref[...] = cp_ref[...] + dc_ref[0] + dc_ref[1]


def _node_mlp(nm, x, w3, b3, w4, b4, cp, dc):
    n = x.shape[0]
    bn = 1000
    return pl.pallas_call(
        _node_body,
        grid=(n // bn,),
        in_specs=[
            pl.BlockSpec((bn, D), lambda i: (i, 0)),
            pl.BlockSpec((bn, D), lambda i: (i, 0)),
            pl.BlockSpec((D, D), lambda i: (0, 0)),
            pl.BlockSpec((1, D), lambda i: (0, 0)),
            pl.BlockSpec((D, D), lambda i: (0, 0)),
            pl.BlockSpec((1, D), lambda i: (0, 0)),
            pl.BlockSpec((bn, CW), lambda i: (i, 0)),
            pl.BlockSpec((2, bn, CW), lambda i: (0, i, 0)),
        ],
        out_specs=[
            pl.BlockSpec((bn, D), lambda i: (i, 0)),
            pl.BlockSpec((bn, CW), lambda i: (i, 0)),
        ],
        out_shape=[
            jax.ShapeDtypeStruct((n, D), jnp.float32),
            jax.ShapeDtypeStruct((n, CW), jnp.float32),
        ],
    )(nm, x, w3, b3.reshape(1, D), w4, b4.reshape(1, D), cp, dc)


def kernel(x, coord, edge_index, W1, b1, W2, b2, W3, b3, W4, b4, W5, b5):
    n, d = x.shape
    row = edge_index[0].astype(jnp.int32)
    col = edge_index[1].astype(jnp.int32)
    w1r = W1[:d]
    w1c = W1[d:2 * d]
    w1d = W1[2 * d].reshape(1, d)
    cpw = jnp.pad(coord, ((0, 0), (0, CW - 3)))
    pr, pc = _project(x, w1r, w1c, b1)
    gr, gc, dx = _gather_combine(_pack_bf16(pr), _pack_bf16(pc),
                                 cpw, -cpw, row, col)
    msg, cu = _edge_mlp(gr, gc, dx, w1d, W2, b2, W5.reshape(1, d), b5)
    np_ = ((n + 2047) // 2048) * 2048
    nm, dc = _scatter_combine(msg, cu, row, np_)
    x_out, co = _node_mlp(nm, x, W3, b3, W4, b4, cpw, dc)
    return (x_out, co[:, :3])


# speedup vs baseline: 3.0859x; 1.1200x over previous
import functools

import jax
import jax.numpy as jnp
from jax import lax
from jax.experimental import pallas as pl
from jax.experimental.pallas import tpu as pltpu
from jax.experimental.pallas import tpu_sc as plsc

D = 512
CW = 128
DI = D // 2
WI = DI + CW
CP = 16
NCORES = 2
NSUB = 16
NW = NCORES * NSUB


def _proj_body(x_ref, w1r_ref, w1c_ref, b1_ref, pr_ref, pc_ref):
    xb = x_ref[...]
    pr = jnp.dot(xb, w1r_ref[...],
                 preferred_element_type=jnp.float32) + b1_ref[...]
    pc = jnp.dot(xb, w1c_ref[...], preferred_element_type=jnp.float32)
    pr_ref[...] = pr.astype(jnp.bfloat16)
    pc_ref[...] = pc.astype(jnp.bfloat16)


def _project(x, w1r, w1c, b1):
    n = x.shape[0]
    bn = 1000
    return pl.pallas_call(
        _proj_body,
        grid=(n // bn,),
        in_specs=[
            pl.BlockSpec((bn, D), lambda i: (i, 0)),
            pl.BlockSpec((D, D), lambda i: (0, 0)),
            pl.BlockSpec((D, D), lambda i: (0, 0)),
            pl.BlockSpec((1, D), lambda i: (0, 0)),
        ],
        out_specs=[
            pl.BlockSpec((bn, D), lambda i: (i, 0)),
            pl.BlockSpec((bn, D), lambda i: (i, 0)),
        ],
        out_shape=[
            jax.ShapeDtypeStruct((n, D), jnp.bfloat16),
            jax.ShapeDtypeStruct((n, D), jnp.bfloat16),
        ],
    )(x, w1r, w1c, b1.reshape(1, D))


def _pack_bf16(v):
    k = v.shape[-1] // 2
    pairs = jnp.stack([v[..., :k], v[..., k:]], axis=-1)
    return jax.lax.bitcast_convert_type(pairs, jnp.int32)


def _gather_combine(trf, tcf, cpw, ncpw, row, col):
    e = row.shape[0]
    epw = e // NW
    cb = 40
    nch = epw // cb
    mesh = plsc.VectorSubcoreMesh(core_axis_name="c", subcore_axis_name="s")

    @functools.partial(
        pl.kernel,
        out_type=[jax.ShapeDtypeStruct((e // cb, cb, DI), jnp.int32),
                  jax.ShapeDtypeStruct((e // cb, cb, DI), jnp.int32),
                  jax.ShapeDtypeStruct((e // cb, cb, CW), jnp.float32)],
        mesh=mesh,
        scratch_types=[
            pltpu.VMEM((cb,), jnp.int32),
            pltpu.VMEM((cb,), jnp.int32),
            pltpu.VMEM((cb,), jnp.int32),
            pltpu.VMEM((cb,), jnp.int32),
            pltpu.VMEM((cb, DI), jnp.int32),
            pltpu.VMEM((cb, DI), jnp.int32),
            pltpu.VMEM((cb, DI), jnp.int32),
            pltpu.VMEM((cb, DI), jnp.int32),
            pltpu.VMEM((cb, CW), jnp.float32),
            pltpu.VMEM((cb, CW), jnp.float32),
            pltpu.VMEM((cb, CW), jnp.float32),
            pltpu.VMEM((cb, CW), jnp.float32),
            pltpu.SemaphoreType.DMA,
            pltpu.SemaphoreType.DMA,
            pltpu.SemaphoreType.DMA,
            pltpu.SemaphoreType.DMA,
        ],
    )
    def k(trf_hbm, tcf_hbm, cpw_hbm, ncpw_hbm, row_hbm, col_hbm,
          gr_hbm, gc_hbm, dx_hbm,
          idxr0, idxr1, idxc0, idxc1, br0, br1, bc0, bc1,
          cr0, cr1, cc0, cc1, gs0, gs1, ws0, ws1):
        idxr = (idxr0, idxr1)
        idxc = (idxc0, idxc1)
        br = (br0, br1)
        bc = (bc0, bc1)
        cr = (cr0, cr1)
        cc = (cc0, cc1)
        gs = (gs0, gs1)
        ws = (ws0, ws1)
        wid = lax.axis_index("s") * NCORES + lax.axis_index("c")
        base0 = wid * epw

        def gfire(ci, s):
            base = base0 + ci * cb
            pltpu.sync_copy(row_hbm.at[pl.ds(base, cb)], idxr[s])
            pltpu.sync_copy(col_hbm.at[pl.ds(base, cb)], idxc[s])
            pltpu.async_copy(trf_hbm.at[idxr[s]], br[s], gs[s])
            pltpu.async_copy(tcf_hbm.at[idxc[s]], bc[s], gs[s])
            pltpu.async_copy(cpw_hbm.at[idxr[s]], cr[s], gs[s])
            pltpu.async_copy(ncpw_hbm.at[idxc[s]], cc[s], gs[s])

        def gwait(s):
            pltpu.make_async_copy(trf_hbm.at[idxr[s]], br[s], gs[s]).wait()
            pltpu.make_async_copy(tcf_hbm.at[idxc[s]], bc[s], gs[s]).wait()
            pltpu.make_async_copy(cpw_hbm.at[idxr[s]], cr[s], gs[s]).wait()
            pltpu.make_async_copy(ncpw_hbm.at[idxc[s]], cc[s], gs[s]).wait()

        def add(s):
            def rowbody(r, acc):
                sl = pl.ds(0, 16)
                cr[s][r, sl] = cr[s][r, sl] + cc[s][r, sl]
                return acc
            lax.fori_loop(0, cb, rowbody, 0)

        def wfire(ci, s):
            gci = wid * nch + ci
            pltpu.async_copy(br[s], gr_hbm.at[gci], ws[s])
            pltpu.async_copy(bc[s], gc_hbm.at[gci], ws[s])
            pltpu.async_copy(cr[s], dx_hbm.at[gci], ws[s])

        def wwait(s):
            pltpu.make_async_copy(br[s], gr_hbm.at[0], ws[s]).wait()
            pltpu.make_async_copy(bc[s], gc_hbm.at[0], ws[s]).wait()
            pltpu.make_async_copy(cr[s], dx_hbm.at[0], ws[s]).wait()

        gfire(0, 0)

        def pair(pi, carry):
            for b in range(2):
                ci = 2 * pi + b
                s = b
                so = 1 - b

                @pl.when(ci > 0)
                def _():
                    wwait(so)

                gfire(ci + 1, so)
                gwait(s)
                add(s)
                wfire(ci, s)
            return carry

        lax.fori_loop(0, (nch - 1) // 2, pair, 0)
        wwait(1)
        gwait(0)
        add(0)
        wfire(nch - 1, 0)
        wwait(0)

    return k(trf, tcf, cpw, ncpw, row, col)


def _unpack_add(wr, wc):
    lo = jax.lax.bitcast_convert_type(wr << 16, jnp.float32) \
        + jax.lax.bitcast_convert_type(wc << 16, jnp.float32)
    hi = jax.lax.bitcast_convert_type(wr & jnp.int32(-65536), jnp.float32) \
        + jax.lax.bitcast_convert_type(wc & jnp.int32(-65536), jnp.float32)
    return jnp.concatenate([lo, hi], axis=-1)


def _edge_body(gr_ref, gc_ref, dx_ref, w1d_ref, w2_ref, b2_ref, w5_ref,
               b5_ref, msg_ref, cu_ref):
    bc, cb, _ = gr_ref.shape
    be = bc * cb
    gb = _unpack_add(gr_ref[...], gc_ref[...]).reshape(be, D)
    dx = dx_ref[...].reshape(be, CW)
    dist = jnp.sqrt(jnp.sum(dx * dx, axis=1, keepdims=True))
    h = gb + dist * w1d_ref[...]
    h = h * jax.nn.sigmoid(h)
    msg = jnp.dot(h.astype(jnp.bfloat16), w2_ref[...],
                  preferred_element_type=jnp.float32) + b2_ref[...]
    msg_ref[...] = msg
    cw = jax.nn.sigmoid(
        jnp.sum(msg * w5_ref[...], axis=1, keepdims=True) + b5_ref[...])
    cu_ref[...] = dx * cw


def _edge_mlp(gr3, gc3, dx3, w1d, w2, b2, w5t, b5):
    nchk, cb, _ = gr3.shape
    e = nchk * cb
    be = 800
    bc = be // cb
    return pl.pallas_call(
        _edge_body,
        grid=(e // be,),
        in_specs=[
            pl.BlockSpec((bc, cb, DI), lambda i: (i, 0, 0)),
            pl.BlockSpec((bc, cb, DI), lambda i: (i, 0, 0)),
            pl.BlockSpec((bc, cb, CW), lambda i: (i, 0, 0)),
            pl.BlockSpec((1, D), lambda i: (0, 0)),
            pl.BlockSpec((D, D), lambda i: (0, 0)),
            pl.BlockSpec((1, D), lambda i: (0, 0)),
            pl.BlockSpec((1, D), lambda i: (0, 0)),
            pl.BlockSpec((1, 1), lambda i: (0, 0)),
        ],
        out_specs=[
            pl.BlockSpec((be, D), lambda i: (i, 0)),
            pl.BlockSpec((be, CW), lambda i: (i, 0)),
        ],
        out_shape=[
            jax.ShapeDtypeStruct((e, D), jnp.float32),
            jax.ShapeDtypeStruct((e, CW), jnp.float32),
        ],
    )(gr3, gc3, dx3, w1d, w2.astype(jnp.bfloat16), b2.reshape(1, D), w5t,
      b5.reshape(1, 1))


def _scatter_combine(msg, cu, row, np_):
    e = msg.shape[0]
    ept = e // NSUB
    bd = 80
    nch = ept // bd
    bd2 = 40
    epth = e // NW
    nch2 = epth // bd2
    rpt = np_ // NSUB
    fs = 128
    zr = 64
    mesh = plsc.VectorSubcoreMesh(core_axis_name="c", subcore_axis_name="s")

    @functools.partial(
        pl.kernel,
        out_type=[jax.ShapeDtypeStruct((np_, D), jnp.float32),
                  jax.ShapeDtypeStruct((2, np_, CW), jnp.float32)],
        mesh=mesh,
        scratch_types=[
            [[pltpu.VMEM((bd,), jnp.int32), pltpu.VMEM((bd2,), jnp.int32)]
             for _ in range(2)],
            pltpu.VMEM((bd, fs), jnp.float32),
            pltpu.VMEM((bd, fs), jnp.float32),
            pltpu.VMEM((zr, fs), jnp.float32),
            pltpu.VMEM_SHARED((np_, fs), jnp.float32),
            pltpu.SemaphoreType.DMA,
            pltpu.SemaphoreType.DMA,
            pltpu.SemaphoreType.DMA,
            pltpu.SemaphoreType.DMA,
        ],
    )
    def k(msg_hbm, cu_hbm, row_hbm, nm_hbm, dc_hbm,
          idxs, mb0, mb1, zbuf, acc, ls0, ls1, ss0, ss1):
        core = lax.axis_index("c")
        tid = lax.axis_index("s")
        r0 = tid * rpt
        mb = (mb0, mb1)
        ls = (ls0, ls1)
        ss = (ss0, ss1)

        def zb(i, c):
            for kk in range(fs // 16):
                zbuf[i, pl.ds(kk * 16, 16)] = jnp.zeros((16,), jnp.float32)
            return c
        lax.fori_loop(0, zr, zb, 0)

        def zero_acc():
            for q in range(rpt // zr):
                pltpu.sync_copy(zbuf, acc.at[pl.ds(r0 + q * zr, zr)])

        def scatter_pass(src_hbm, joff, ebase0, nchp, bdp, ij):
            def lfire(ci, s):
                base = ebase0 + ci * bdp
                pltpu.async_copy(row_hbm.at[pl.ds(base, bdp)],
                                 idxs[s][ij], ls[s])
                pltpu.async_copy(
                    src_hbm.at[pl.ds(base, bdp), pl.ds(joff, fs)],
                    mb[s].at[pl.ds(0, bdp)], ls[s])

            def lwait(s):
                pltpu.make_async_copy(row_hbm.at[pl.ds(ebase0, bdp)],
                                      idxs[s][ij], ls[s]).wait()
                pltpu.make_async_copy(
                    src_hbm.at[pl.ds(ebase0, bdp), pl.ds(joff, fs)],
                    mb[s].at[pl.ds(0, bdp)], ls[s]).wait()

            def sfire(s):
                pltpu.async_copy(mb[s].at[pl.ds(0, bdp)],
                                 acc.at[idxs[s][ij]], ss[s], add=True)

            def swait(s):
                pltpu.make_async_copy(mb[s].at[pl.ds(0, bdp)],
                                      acc.at[idxs[s][ij]], ss[s]).wait()

            lfire(0, 0)

            def pair(pi, carry):
                for b in range(2):
                    ci = 2 * pi + b
                    s = b
                    so = 1 - b
                    lwait(s)
                    sfire(s)

                    @pl.when(ci > 0)
                    def _():
                        swait(so)

                    @pl.when(ci + 1 < nchp)
                    def _():
                        lfire(ci + 1, so)
                return carry

            lax.fori_loop(0, nchp // 2, pair, 0)
            if nchp % 2 == 1:
                lwait(0)
                sfire(0)
                swait(1)
                swait(0)
            else:
                swait(1)

        zero_acc()
        plsc.subcore_barrier()

        for p in range(2):
            joff = (core * 2 + p) * fs
            scatter_pass(msg_hbm, joff, tid * ept, nch, bd, 0)
            plsc.subcore_barrier()
            pltpu.sync_copy(acc.at[pl.ds(r0, rpt)],
                            nm_hbm.at[pl.ds(r0, rpt), pl.ds(joff, fs)])
            zero_acc()
            plsc.subcore_barrier()

        scatter_pass(cu_hbm, 0, core * (e // 2) + tid * epth, nch2, bd2, 1)
        plsc.subcore_barrier()
        pltpu.sync_copy(acc.at[pl.ds(r0, rpt)],
                        dc_hbm.at[core, pl.ds(r0, rpt)])

    return k(msg, cu, row)


def _node_body(nm_ref, x_ref, w3_ref, b3_ref, w4_ref, b4_ref,
               cp_ref, dc_ref, xo_ref, co_ref):
    t = jnp.dot(nm_ref[...], w3_ref[...],
                preferred_element_type=jnp.float32) + b3_ref[...]
    t = t * jax.nn.sigmoid(t)
    xo_ref[...] = x_ref[...] + jnp.dot(
        t, w4_ref[...], preferred_element_type=jnp.float32) + b4_ref[...]
    co_ref[...] = cp_ref[...] + dc_ref[0] + dc_ref[1]


def _node_mlp(nm, x, w3, b3, w4, b4, cp, dc):
    n = x.shape[0]
    bn = 1000
    return pl.pallas_call(
        _node_body,
        grid=(n // bn,),
        in_specs=[
            pl.BlockSpec((bn, D), lambda i: (i, 0)),
            pl.BlockSpec((bn, D), lambda i: (i, 0)),
            pl.BlockSpec((D, D), lambda i: (0, 0)),
            pl.BlockSpec((1, D), lambda i: (0, 0)),
            pl.BlockSpec((D, D), lambda i: (0, 0)),
            pl.BlockSpec((1, D), lambda i: (0, 0)),
            pl.BlockSpec((bn, CW), lambda i: (i, 0)),
            pl.BlockSpec((2, bn, CW), lambda i: (0, i, 0)),
        ],
        out_specs=[
            pl.BlockSpec((bn, D), lambda i: (i, 0)),
            pl.BlockSpec((bn, CW), lambda i: (i, 0)),
        ],
        out_shape=[
            jax.ShapeDtypeStruct((n, D), jnp.float32),
            jax.ShapeDtypeStruct((n, CW), jnp.float32),
        ],
    )(nm, x, w3, b3.reshape(1, D), w4, b4.reshape(1, D), cp, dc)


def kernel(x, coord, edge_index, W1, b1, W2, b2, W3, b3, W4, b4, W5, b5):
    n, d = x.shape
    row = edge_index[0].astype(jnp.int32)
    col = edge_index[1].astype(jnp.int32)
    w1r = W1[:d]
    w1c = W1[d:2 * d]
    w1d = W1[2 * d].reshape(1, d)
    cpw = jnp.pad(coord, ((0, 0), (0, CW - 3)))
    pr, pc = _project(x, w1r, w1c, b1)
    gr, gc, dx = _gather_combine(_pack_bf16(pr), _pack_bf16(pc),
                                 cpw, -cpw, row, col)
    msg, cu = _edge_mlp(gr, gc, dx, w1d, W2, b2, W5.reshape(1, d), b5)
    np_ = ((n + 2047) // 2048) * 2048
    nm, dc = _scatter_combine(msg, cu, row, np_)
    x_out, co = _node_mlp(nm, x, W3, b3, W4, b4, cpw, dc)
    return (x_out, co[:, :3])


# speedup vs baseline: 3.2405x; 1.0501x over previous
import functools

import jax
import jax.numpy as jnp
from jax import lax
from jax.experimental import pallas as pl
from jax.experimental.pallas import tpu as pltpu
from jax.experimental.pallas import tpu_sc as plsc

D = 512
CW = 128
DI = D // 2
WI = DI + CW
CP = 16
NCORES = 2
NSUB = 16
NW = NCORES * NSUB


def _proj_body(x_ref, w1r_ref, w1c_ref, b1_ref, pr_ref, pc_ref):
    xb = x_ref[...]
    pr = jnp.dot(xb, w1r_ref[...],
                 preferred_element_type=jnp.float32) + b1_ref[...]
    pc = jnp.dot(xb, w1c_ref[...], preferred_element_type=jnp.float32)
    pr_ref[...] = pr.astype(jnp.bfloat16)
    pc_ref[...] = pc.astype(jnp.bfloat16)


def _project(x, w1r, w1c, b1):
    n = x.shape[0]
    bn = 1000
    return pl.pallas_call(
        _proj_body,
        grid=(n // bn,),
        in_specs=[
            pl.BlockSpec((bn, D), lambda i: (i, 0)),
            pl.BlockSpec((D, D), lambda i: (0, 0)),
            pl.BlockSpec((D, D), lambda i: (0, 0)),
            pl.BlockSpec((1, D), lambda i: (0, 0)),
        ],
        out_specs=[
            pl.BlockSpec((bn, D), lambda i: (i, 0)),
            pl.BlockSpec((bn, D), lambda i: (i, 0)),
        ],
        out_shape=[
            jax.ShapeDtypeStruct((n, D), jnp.bfloat16),
            jax.ShapeDtypeStruct((n, D), jnp.bfloat16),
        ],
    )(x, w1r, w1c, b1.reshape(1, D))


def _pack_bf16(v):
    k = v.shape[-1] // 2
    pairs = jnp.stack([v[..., :k], v[..., k:]], axis=-1)
    return jax.lax.bitcast_convert_type(pairs, jnp.int32)


def _gather_combine(trf, tcf, cpw, ncpw, row, col):
    e = row.shape[0]
    epw = e // NW
    cb = 40
    nch = epw // cb
    mesh = plsc.VectorSubcoreMesh(core_axis_name="c", subcore_axis_name="s")

    @functools.partial(
        pl.kernel,
        out_type=[jax.ShapeDtypeStruct((e // cb, cb, DI), jnp.int32),
                  jax.ShapeDtypeStruct((e // cb, cb, DI), jnp.int32),
                  jax.ShapeDtypeStruct((e // cb, cb, CW), jnp.float32)],
        mesh=mesh,
        scratch_types=[
            pltpu.VMEM((cb,), jnp.int32),
            pltpu.VMEM((cb,), jnp.int32),
            pltpu.VMEM((cb,), jnp.int32),
            pltpu.VMEM((cb,), jnp.int32),
            pltpu.VMEM((cb, DI), jnp.int32),
            pltpu.VMEM((cb, DI), jnp.int32),
            pltpu.VMEM((cb, DI), jnp.int32),
            pltpu.VMEM((cb, DI), jnp.int32),
            pltpu.VMEM((cb, CW), jnp.float32),
            pltpu.VMEM((cb, CW), jnp.float32),
            pltpu.VMEM((cb, CW), jnp.float32),
            pltpu.VMEM((cb, CW), jnp.float32),
            pltpu.SemaphoreType.DMA,
            pltpu.SemaphoreType.DMA,
            pltpu.SemaphoreType.DMA,
            pltpu.SemaphoreType.DMA,
        ],
    )
    def k(trf_hbm, tcf_hbm, cpw_hbm, ncpw_hbm, row_hbm, col_hbm,
          gr_hbm, gc_hbm, dx_hbm,
          idxr0, idxr1, idxc0, idxc1, br0, br1, bc0, bc1,
          cr0, cr1, cc0, cc1, gs0, gs1, ws0, ws1):
        idxr = (idxr0, idxr1)
        idxc = (idxc0, idxc1)
        br = (br0, br1)
        bc = (bc0, bc1)
        cr = (cr0, cr1)
        cc = (cc0, cc1)
        gs = (gs0, gs1)
        ws = (ws0, ws1)
        wid = lax.axis_index("s") * NCORES + lax.axis_index("c")
        base0 = wid * epw

        def gfire(ci, s):
            base = base0 + ci * cb
            pltpu.sync_copy(row_hbm.at[pl.ds(base, cb)], idxr[s])
            pltpu.sync_copy(col_hbm.at[pl.ds(base, cb)], idxc[s])
            pltpu.async_copy(trf_hbm.at[idxr[s]], br[s], gs[s])
            pltpu.async_copy(tcf_hbm.at[idxc[s]], bc[s], gs[s])
            pltpu.async_copy(cpw_hbm.at[idxr[s]], cr[s], gs[s])
            pltpu.async_copy(ncpw_hbm.at[idxc[s]], cc[s], gs[s])

        def gwait(s):
            pltpu.make_async_copy(trf_hbm.at[idxr[s]], br[s], gs[s]).wait()
            pltpu.make_async_copy(tcf_hbm.at[idxc[s]], bc[s], gs[s]).wait()
            pltpu.make_async_copy(cpw_hbm.at[idxr[s]], cr[s], gs[s]).wait()
            pltpu.make_async_copy(ncpw_hbm.at[idxc[s]], cc[s], gs[s]).wait()

        def add(s):
            def rowbody(r, acc):
                sl = pl.ds(0, 16)
                cr[s][r, sl] = cr[s][r, sl] + cc[s][r, sl]
                return acc
            lax.fori_loop(0, cb, rowbody, 0)

        def wfire(ci, s):
            gci = wid * nch + ci
            pltpu.async_copy(br[s], gr_hbm.at[gci], ws[s])
            pltpu.async_copy(bc[s], gc_hbm.at[gci], ws[s])
            pltpu.async_copy(cr[s], dx_hbm.at[gci], ws[s])

        def wwait(s):
            pltpu.make_async_copy(br[s], gr_hbm.at[0], ws[s]).wait()
            pltpu.make_async_copy(bc[s], gc_hbm.at[0], ws[s]).wait()
            pltpu.make_async_copy(cr[s], dx_hbm.at[0], ws[s]).wait()

        gfire(0, 0)

        def pair(pi, carry):
            for b in range(2):
                ci = 2 * pi + b
                s = b
                so = 1 - b

                @pl.when(ci > 0)
                def _():
                    wwait(so)

                @pl.when(ci + 1 < nch)
                def _():
                    gfire(ci + 1, so)

                gwait(s)
                add(s)
                wfire(ci, s)
            return carry

        lax.fori_loop(0, nch // 2, pair, 0)
        if nch % 2 == 1:
            wwait(1)
            gwait(0)
            add(0)
            wfire(nch - 1, 0)
            wwait(0)
        else:
            wwait(1)

    return k(trf, tcf, cpw, ncpw, row, col)


def _unpack_add(wr, wc):
    lo = jax.lax.bitcast_convert_type(wr << 16, jnp.float32) \
        + jax.lax.bitcast_convert_type(wc << 16, jnp.float32)
    hi = jax.lax.bitcast_convert_type(wr & jnp.int32(-65536), jnp.float32) \
        + jax.lax.bitcast_convert_type(wc & jnp.int32(-65536), jnp.float32)
    return jnp.concatenate([lo, hi], axis=-1)


def _edge_body(gr_ref, gc_ref, dx_ref, w1d_ref, w2_ref, b2_ref, w5_ref,
               b5_ref, msg_ref, cu_ref):
    bc, cb, _ = gr_ref.shape
    be = bc * cb
    gb = _unpack_add(gr_ref[...], gc_ref[...]).reshape(be, D)
    dx = dx_ref[...].reshape(be, CW)
    dist = jnp.sqrt(jnp.sum(dx * dx, axis=1, keepdims=True))
    h = gb + dist * w1d_ref[...]
    h = h * jax.nn.sigmoid(h)
    msg = jnp.dot(h.astype(jnp.bfloat16), w2_ref[...],
                  preferred_element_type=jnp.float32) + b2_ref[...]
    msg_ref[...] = msg
    cw = jax.nn.sigmoid(
        jnp.sum(msg * w5_ref[...], axis=1, keepdims=True) + b5_ref[...])
    cu_ref[...] = dx * cw


def _edge_mlp(gr3, gc3, dx3, w1d, w2, b2, w5t, b5):
    nchk, cb, _ = gr3.shape
    e = nchk * cb
    be = 800
    bc = be // cb
    return pl.pallas_call(
        _edge_body,
        grid=(e // be,),
        in_specs=[
            pl.BlockSpec((bc, cb, DI), lambda i: (i, 0, 0)),
            pl.BlockSpec((bc, cb, DI), lambda i: (i, 0, 0)),
            pl.BlockSpec((bc, cb, CW), lambda i: (i, 0, 0)),
            pl.BlockSpec((1, D), lambda i: (0, 0)),
            pl.BlockSpec((D, D), lambda i: (0, 0)),
            pl.BlockSpec((1, D), lambda i: (0, 0)),
            pl.BlockSpec((1, D), lambda i: (0, 0)),
            pl.BlockSpec((1, 1), lambda i: (0, 0)),
        ],
        out_specs=[
            pl.BlockSpec((be, D), lambda i: (i, 0)),
            pl.BlockSpec((be, CW), lambda i: (i, 0)),
        ],
        out_shape=[
            jax.ShapeDtypeStruct((e, D), jnp.float32),
            jax.ShapeDtypeStruct((e, CW), jnp.float32),
        ],
    )(gr3, gc3, dx3, w1d, w2.astype(jnp.bfloat16), b2.reshape(1, D), w5t,
      b5.reshape(1, 1))


def _scatter_combine(msg_a, cu_a, row_a, msg_b, cu_b, row_b, np_):
    ea = msg_a.shape[0]
    eb = msg_b.shape[0]
    bd = 80
    bd2 = 40
    rpt = np_ // NSUB
    fs = 128
    zr = 64
    mesh = plsc.VectorSubcoreMesh(core_axis_name="c", subcore_axis_name="s")

    @functools.partial(
        pl.kernel,
        out_type=[jax.ShapeDtypeStruct((np_, D), jnp.float32),
                  jax.ShapeDtypeStruct((2, np_, CW), jnp.float32)],
        mesh=mesh,
        scratch_types=[
            [[pltpu.VMEM((bd,), jnp.int32), pltpu.VMEM((bd2,), jnp.int32)]
             for _ in range(2)],
            pltpu.VMEM((bd, fs), jnp.float32),
            pltpu.VMEM((bd, fs), jnp.float32),
            pltpu.VMEM((zr, fs), jnp.float32),
            pltpu.VMEM_SHARED((np_, fs), jnp.float32),
            pltpu.SemaphoreType.DMA,
            pltpu.SemaphoreType.DMA,
            pltpu.SemaphoreType.DMA,
            pltpu.SemaphoreType.DMA,
        ],
    )
    def k(msga_hbm, cua_hbm, rowa_hbm, msgb_hbm, cub_hbm, rowb_hbm,
          nm_hbm, dc_hbm, idxs, mb0, mb1, zbuf, acc, ls0, ls1, ss0, ss1):
        core = lax.axis_index("c")
        tid = lax.axis_index("s")
        r0 = tid * rpt
        mb = (mb0, mb1)
        ls = (ls0, ls1)
        ss = (ss0, ss1)

        def zb(i, c):
            for kk in range(fs // 16):
                zbuf[i, pl.ds(kk * 16, 16)] = jnp.zeros((16,), jnp.float32)
            return c
        lax.fori_loop(0, zr, zb, 0)

        def zero_acc():
            for q in range(rpt // zr):
                pltpu.sync_copy(zbuf, acc.at[pl.ds(r0 + q * zr, zr)])

        def scatter_pass(src_hbm, row_hbm, joff, ebase0, nchp, bdp, ij):
            def lfire(ci, s):
                base = ebase0 + ci * bdp
                pltpu.async_copy(row_hbm.at[pl.ds(base, bdp)],
                                 idxs[s][ij], ls[s])
                pltpu.async_copy(
                    src_hbm.at[pl.ds(base, bdp), pl.ds(joff, fs)],
                    mb[s].at[pl.ds(0, bdp)], ls[s])

            def lwait(s):
                pltpu.make_async_copy(row_hbm.at[pl.ds(ebase0, bdp)],
                                      idxs[s][ij], ls[s]).wait()
                pltpu.make_async_copy(
                    src_hbm.at[pl.ds(ebase0, bdp), pl.ds(joff, fs)],
                    mb[s].at[pl.ds(0, bdp)], ls[s]).wait()

            def sfire(s):
                pltpu.async_copy(mb[s].at[pl.ds(0, bdp)],
                                 acc.at[idxs[s][ij]], ss[s], add=True)

            def swait(s):
                pltpu.make_async_copy(mb[s].at[pl.ds(0, bdp)],
                                      acc.at[idxs[s][ij]], ss[s]).wait()

            lfire(0, 0)

            def pair(pi, carry):
                for b in range(2):
                    ci = 2 * pi + b
                    s = b
                    so = 1 - b
                    lwait(s)
                    sfire(s)

                    @pl.when(ci > 0)
                    def _():
                        swait(so)

                    @pl.when(ci + 1 < nchp)
                    def _():
                        lfire(ci + 1, so)
                return carry

            lax.fori_loop(0, nchp // 2, pair, 0)
            if nchp % 2 == 1:
                lwait(0)
                sfire(0)
                swait(1)
                swait(0)
            else:
                swait(1)

        zero_acc()
        plsc.subcore_barrier()

        for p in range(2):
            joff = (core * 2 + p) * fs
            scatter_pass(msga_hbm, rowa_hbm, joff,
                         tid * (ea // NSUB), (ea // NSUB) // bd, bd, 0)
            scatter_pass(msgb_hbm, rowb_hbm, joff,
                         tid * (eb // NSUB), (eb // NSUB) // bd, bd, 0)
            plsc.subcore_barrier()
            pltpu.sync_copy(acc.at[pl.ds(r0, rpt)],
                            nm_hbm.at[pl.ds(r0, rpt), pl.ds(joff, fs)])
            zero_acc()
            plsc.subcore_barrier()

        scatter_pass(cua_hbm, rowa_hbm, 0,
                     core * (ea // 2) + tid * (ea // NW),
                     (ea // NW) // bd2, bd2, 1)
        scatter_pass(cub_hbm, rowb_hbm, 0,
                     core * (eb // 2) + tid * (eb // NW),
                     (eb // NW) // bd2, bd2, 1)
        plsc.subcore_barrier()
        pltpu.sync_copy(acc.at[pl.ds(r0, rpt)],
                        dc_hbm.at[core, pl.ds(r0, rpt)])

    return k(msg_a, cu_a, row_a, msg_b, cu_b, row_b)


def _node_body(nm_ref, x_ref, w3_ref, b3_ref, w4_ref, b4_ref,
               cp_ref, dc_ref, xo_ref, co_ref):
    t = jnp.dot(nm_ref[...], w3_ref[...],
                preferred_element_type=jnp.float32) + b3_ref[...]
    t = t * jax.nn.sigmoid(t)
    xo_ref[...] = x_ref[...] + jnp.dot(
        t, w4_ref[...], preferred_element_type=jnp.float32) + b4_ref[...]
    co_ref[...] = cp_ref[...] + dc_ref[0] + dc_ref[1]


def _node_mlp(nm, x, w3, b3, w4, b4, cp, dc):
    n = x.shape[0]
    bn = 1000
    return pl.pallas_call(
        _node_body,
        grid=(n // bn,),
        in_specs=[
            pl.BlockSpec((bn, D), lambda i: (i, 0)),
            pl.BlockSpec((bn, D), lambda i: (i, 0)),
            pl.BlockSpec((D, D), lambda i: (0, 0)),
            pl.BlockSpec((1, D), lambda i: (0, 0)),
            pl.BlockSpec((D, D), lambda i: (0, 0)),
            pl.BlockSpec((1, D), lambda i: (0, 0)),
            pl.BlockSpec((bn, CW), lambda i: (i, 0)),
            pl.BlockSpec((2, bn, CW), lambda i: (0, i, 0)),
        ],
        out_specs=[
            pl.BlockSpec((bn, D), lambda i: (i, 0)),
            pl.BlockSpec((bn, CW), lambda i: (i, 0)),
        ],
        out_shape=[
            jax.ShapeDtypeStruct((n, D), jnp.float32),
            jax.ShapeDtypeStruct((n, CW), jnp.float32),
        ],
    )(nm, x, w3, b3.reshape(1, D), w4, b4.reshape(1, D), cp, dc)


def kernel(x, coord, edge_index, W1, b1, W2, b2, W3, b3, W4, b4, W5, b5):
    n, d = x.shape
    row = edge_index[0].astype(jnp.int32)
    col = edge_index[1].astype(jnp.int32)
    w1r = W1[:d]
    w1c = W1[d:2 * d]
    w1d = W1[2 * d].reshape(1, d)
    cpw = jnp.pad(coord, ((0, 0), (0, CW - 3)))
    pr, pc = _project(x, w1r, w1c, b1)
    trf = _pack_bf16(pr)
    tcf = _pack_bf16(pc)
    ncpw = -cpw
    e = row.shape[0]
    ea = e * 3 // 5
    w5t = W5.reshape(1, d)
    gr_a, gc_a, dx_a = _gather_combine(trf, tcf, cpw, ncpw,
                                       row[:ea], col[:ea])
    gr_b, gc_b, dx_b = _gather_combine(trf, tcf, cpw, ncpw,
                                       row[ea:], col[ea:])
    msg_a, cu_a = _edge_mlp(gr_a, gc_a, dx_a, w1d, W2, b2, w5t, b5)
    msg_b, cu_b = _edge_mlp(gr_b, gc_b, dx_b, w1d, W2, b2, w5t, b5)
    np_ = ((n + 2047) // 2048) * 2048
    nm, dc = _scatter_combine(msg_a, cu_a, row[:ea],
                              msg_b, cu_b, row[ea:], np_)
    x_out, co = _node_mlp(nm, x, W3, b3, W4, b4, cpw, dc)
    return (x_out, co[:, :3])
